# Initial kernel scaffold; baseline (speedup 1.0000x reference)
import jax
import jax.numpy as jnp
from jax.experimental import pallas as pl


def kernel(coordinates, edge_attr, src, dst, non_fictitious, W1_src, b1_src, W2_src, b2_src, W1_dst, b1_dst, W2_dst, b2_dst):
    raise NotImplementedError("write your pallas kernel here")


# speedup vs baseline: 2.5754x; 2.5754x over previous
import functools

import jax
import jax.numpy as jnp
from jax import lax
from jax.experimental import pallas as pl
from jax.experimental.pallas import tpu as pltpu
from jax.experimental.pallas import tpu_sc as plsc

N = 10000
E = 320000
D = 128
D_EDGE = 16
HID2 = 256
OUT = 128

NC = 2
NS = 16
NW = NC * NS
EPW = E // NW
CHUNK = 80
NCHUNK = EPW // CHUNK
N_PAD = 10240
ROWS_PER_TILE = N_PAD // NS


def _k1_body(coords, w1a, w1b, p1, p2):
    c = coords[...]
    p1[...] = jnp.dot(c, w1a[...], preferred_element_type=jnp.float32)
    p2[...] = jnp.dot(c, w1b[...], preferred_element_type=jnp.float32)


def _project(coords, w1a, w1b):
    blk = 1000
    return pl.pallas_call(
        _k1_body,
        grid=(N // blk,),
        in_specs=[
            pl.BlockSpec((blk, D), lambda i: (i, 0)),
            pl.BlockSpec((D, HID2), lambda i: (0, 0)),
            pl.BlockSpec((D, HID2), lambda i: (0, 0)),
        ],
        out_specs=[
            pl.BlockSpec((blk, HID2), lambda i: (i, 0)),
            pl.BlockSpec((blk, HID2), lambda i: (i, 0)),
        ],
        out_shape=[
            jax.ShapeDtypeStruct((N, HID2), jnp.float32),
            jax.ShapeDtypeStruct((N, HID2), jnp.float32),
        ],
    )(coords, w1a, w1b)


def _k2_body(p1_hbm, p2_hbm, src_hbm, dst_hbm, g1_hbm, g2_hbm,
             idx1_v, idx2_v, buf1_v, buf2_v):
    wid = lax.axis_index("s") * NC + lax.axis_index("c")
    base = pl.multiple_of(wid * EPW, 8)

    def step(k, carry):
        off = pl.multiple_of(base + k * CHUNK, 8)
        pltpu.sync_copy(src_hbm.at[pl.ds(off, CHUNK)], idx1_v)
        pltpu.sync_copy(dst_hbm.at[pl.ds(off, CHUNK)], idx2_v)
        pltpu.sync_copy(p1_hbm.at[idx1_v], buf1_v)
        pltpu.sync_copy(p2_hbm.at[idx2_v], buf2_v)
        pltpu.sync_copy(buf1_v, g1_hbm.at[pl.ds(off, CHUNK)])
        pltpu.sync_copy(buf2_v, g2_hbm.at[pl.ds(off, CHUNK)])
        return carry

    lax.fori_loop(0, NCHUNK, step, 0)


def _gather(p1, p2, src, dst):
    mesh = plsc.VectorSubcoreMesh(core_axis_name="c", subcore_axis_name="s")
    f = functools.partial(
        pl.kernel,
        out_type=[
            jax.ShapeDtypeStruct((E, HID2), jnp.float32),
            jax.ShapeDtypeStruct((E, HID2), jnp.float32),
        ],
        mesh=mesh,
        scratch_types=[
            pltpu.VMEM((CHUNK,), jnp.int32),
            pltpu.VMEM((CHUNK,), jnp.int32),
            pltpu.VMEM((CHUNK, HID2), jnp.float32),
            pltpu.VMEM((CHUNK, HID2), jnp.float32),
        ],
    )(_k2_body)
    return f(p1, p2, src, dst)


def _k3_body(ea, g1, g2, mask, w1e, b1, w2, b2, out):
    q = jnp.dot(ea[...], w1e[...], preferred_element_type=jnp.float32)
    q = q + g1[...] + g2[...] + b1[...]
    h = jnp.maximum(q, 0.0) * mask[...]
    out[...] = (jnp.dot(h, w2[...], preferred_element_type=jnp.float32)
                + mask[...] * b2[...])


def _edge_mlp(edge_attr, g1, g2, mask2d, w1e, b1, w2, b2):
    blk = 2000
    return pl.pallas_call(
        _k3_body,
        grid=(E // blk,),
        in_specs=[
            pl.BlockSpec((blk, D_EDGE), lambda i: (i, 0)),
            pl.BlockSpec((blk, HID2), lambda i: (i, 0)),
            pl.BlockSpec((blk, HID2), lambda i: (i, 0)),
            pl.BlockSpec((blk, 1), lambda i: (i, 0)),
            pl.BlockSpec((D_EDGE, HID2), lambda i: (0, 0)),
            pl.BlockSpec((1, HID2), lambda i: (0, 0)),
            pl.BlockSpec((HID2, OUT), lambda i: (0, 0)),
            pl.BlockSpec((1, OUT), lambda i: (0, 0)),
        ],
        out_specs=pl.BlockSpec((blk, OUT), lambda i: (i, 0)),
        out_shape=jax.ShapeDtypeStruct((E, OUT), jnp.float32),
    )(edge_attr, g1, g2, mask2d, w1e, b1, w2, b2)


def _k4_body(dst_hbm, msg_hbm, zeros_hbm, pacc_hbm,
             idx_v, rows_v, acc_spmem):
    c = lax.axis_index("c")
    s = lax.axis_index("s")
    zoff = pl.multiple_of(s * ROWS_PER_TILE, 8)
    pltpu.sync_copy(zeros_hbm, acc_spmem.at[pl.ds(zoff, ROWS_PER_TILE)])
    plsc.subcore_barrier()

    base = pl.multiple_of((c * NS + s) * EPW, 8)

    def step(k, carry):
        off = pl.multiple_of(base + k * CHUNK, 8)
        pltpu.sync_copy(dst_hbm.at[pl.ds(off, CHUNK)], idx_v)
        pltpu.sync_copy(msg_hbm.at[pl.ds(off, CHUNK)], rows_v)
        pltpu.sync_copy(rows_v, acc_spmem.at[idx_v], add=True)
        return carry

    lax.fori_loop(0, NCHUNK, step, 0)
    plsc.subcore_barrier()
    pltpu.sync_copy(acc_spmem.at[pl.ds(zoff, ROWS_PER_TILE)],
                    pacc_hbm.at[c, pl.ds(zoff, ROWS_PER_TILE)])


def _scatter(dst, msg, zeros):
    mesh = plsc.VectorSubcoreMesh(core_axis_name="c", subcore_axis_name="s")
    f = functools.partial(
        pl.kernel,
        out_type=jax.ShapeDtypeStruct((NC, N_PAD, OUT), jnp.float32),
        mesh=mesh,
        scratch_types=[
            pltpu.VMEM((CHUNK,), jnp.int32),
            pltpu.VMEM((CHUNK, OUT), jnp.float32),
            pltpu.VMEM_SHARED((N_PAD, OUT), jnp.float32),
        ],
    )(_k4_body)
    return f(dst, msg, zeros)


def _k5_body(pacc, out):
    p = pacc[...]
    out[...] = jnp.maximum(p[0] + p[1], 0.0)


def _finish(pacc):
    blk = 1000
    return pl.pallas_call(
        _k5_body,
        grid=(N // blk,),
        in_specs=[pl.BlockSpec((NC, blk, OUT), lambda i: (0, i, 0))],
        out_specs=pl.BlockSpec((blk, OUT), lambda i: (i, 0)),
        out_shape=jax.ShapeDtypeStruct((N, OUT), jnp.float32),
    )(pacc)


def kernel(coordinates, edge_attr, src, dst, non_fictitious,
           W1_src, b1_src, W2_src, b2_src,
           W1_dst, b1_dst, W2_dst, b2_dst):
    W1c = jnp.concatenate([W1_src, W1_dst], axis=1)
    b1c = jnp.concatenate([b1_src, b1_dst])[None, :]
    W2c = jnp.concatenate([W2_src, W2_dst], axis=0)
    b2c = (b2_src + b2_dst)[None, :]
    W1e = W1c[:D_EDGE]
    W1a = W1c[D_EDGE:D_EDGE + D]
    W1b = W1c[D_EDGE + D:]

    p1, p2 = _project(coordinates, W1a, W1b)
    g1, g2 = _gather(p1, p2, src, dst)
    msg = _edge_mlp(edge_attr, g1, g2, non_fictitious[:, None],
                    W1e, b1c, W2c, b2c)
    zeros = jnp.zeros((ROWS_PER_TILE, OUT), jnp.float32)
    pacc = _scatter(dst, msg, zeros)
    return _finish(pacc)


# speedup vs baseline: 3.1355x; 1.2175x over previous
import functools

import jax
import jax.numpy as jnp
from jax import lax
from jax.experimental import pallas as pl
from jax.experimental.pallas import tpu as pltpu
from jax.experimental.pallas import tpu_sc as plsc

N = 10000
E = 320000
D = 128
D_EDGE = 16
HID2 = 256
OUT = 128

NC = 2
NS = 16
NW = NC * NS
EPW = E // NW
CHUNK = 80
NCHUNK = EPW // CHUNK
N_PAD = 10240
ROWS_PER_TILE = N_PAD // NS


def _pack_bf16_pair(x):
    lo = jax.lax.bitcast_convert_type(
        x[:, :HID2 // 2].astype(jnp.bfloat16), jnp.uint16).astype(jnp.uint32)
    hi = jax.lax.bitcast_convert_type(
        x[:, HID2 // 2:].astype(jnp.bfloat16), jnp.uint16).astype(jnp.uint32)
    return jax.lax.bitcast_convert_type((hi << 16) | lo, jnp.int32)


def _k1_body(coords, w1a, w1b, p1, p2):
    c = coords[...]
    p1[...] = _pack_bf16_pair(
        jnp.dot(c, w1a[...], preferred_element_type=jnp.float32))
    p2[...] = _pack_bf16_pair(
        jnp.dot(c, w1b[...], preferred_element_type=jnp.float32))


def _project(coords, w1a, w1b):
    blk = 1000
    return pl.pallas_call(
        _k1_body,
        grid=(N // blk,),
        in_specs=[
            pl.BlockSpec((blk, D), lambda i: (i, 0)),
            pl.BlockSpec((D, HID2), lambda i: (0, 0)),
            pl.BlockSpec((D, HID2), lambda i: (0, 0)),
        ],
        out_specs=[
            pl.BlockSpec((blk, HID2 // 2), lambda i: (i, 0)),
            pl.BlockSpec((blk, HID2 // 2), lambda i: (i, 0)),
        ],
        out_shape=[
            jax.ShapeDtypeStruct((N, HID2 // 2), jnp.int32),
            jax.ShapeDtypeStruct((N, HID2 // 2), jnp.int32),
        ],
    )(coords, w1a, w1b)


def _k2_body(p1_hbm, p2_hbm, src_hbm, dst_hbm, g1_hbm, g2_hbm,
             idx1_v, idx2_v, buf1_v, buf2_v):
    wid = lax.axis_index("s") * NC + lax.axis_index("c")
    base = pl.multiple_of(wid * EPW, 8)

    def step(k, carry):
        off = pl.multiple_of(base + k * CHUNK, 8)
        pltpu.sync_copy(src_hbm.at[pl.ds(off, CHUNK)], idx1_v)
        pltpu.sync_copy(dst_hbm.at[pl.ds(off, CHUNK)], idx2_v)
        pltpu.sync_copy(p1_hbm.at[idx1_v], buf1_v)
        pltpu.sync_copy(p2_hbm.at[idx2_v], buf2_v)
        pltpu.sync_copy(buf1_v, g1_hbm.at[pl.ds(off, CHUNK)])
        pltpu.sync_copy(buf2_v, g2_hbm.at[pl.ds(off, CHUNK)])
        return carry

    lax.fori_loop(0, NCHUNK, step, 0)


def _gather(p1, p2, src, dst):
    mesh = plsc.VectorSubcoreMesh(core_axis_name="c", subcore_axis_name="s")
    f = functools.partial(
        pl.kernel,
        out_type=[
            jax.ShapeDtypeStruct((E, HID2 // 2), jnp.int32),
            jax.ShapeDtypeStruct((E, HID2 // 2), jnp.int32),
        ],
        mesh=mesh,
        scratch_types=[
            pltpu.VMEM((CHUNK,), jnp.int32),
            pltpu.VMEM((CHUNK,), jnp.int32),
            pltpu.VMEM((CHUNK, HID2 // 2), jnp.int32),
            pltpu.VMEM((CHUNK, HID2 // 2), jnp.int32),
        ],
    )(_k2_body)
    return f(p1, p2, src, dst)


def _unpack_lo(w):
    return jax.lax.bitcast_convert_type(w << 16, jnp.float32)


def _unpack_hi(w):
    return jax.lax.bitcast_convert_type(
        w & jnp.int32(-65536), jnp.float32)


def _k3_body(ea, g1, g2, mask, w1e, b1, w2, b2, out):
    H = HID2 // 2
    a = jnp.dot(ea[...], w1e[...], preferred_element_type=jnp.float32) + b1[...]
    g1w, g2w = g1[...], g2[...]
    m = mask[...]
    q_lo = a[:, :H] + _unpack_lo(g1w) + _unpack_lo(g2w)
    q_hi = a[:, H:] + _unpack_hi(g1w) + _unpack_hi(g2w)
    h_lo = jnp.maximum(q_lo, 0.0) * m
    h_hi = jnp.maximum(q_hi, 0.0) * m
    w2v = w2[...]
    out[...] = (jnp.dot(h_lo, w2v[:H], preferred_element_type=jnp.float32)
                + jnp.dot(h_hi, w2v[H:], preferred_element_type=jnp.float32)
                + m * b2[...])


def _edge_mlp(edge_attr, g1, g2, mask2d, w1e, b1, w2, b2):
    blk = 2000
    return pl.pallas_call(
        _k3_body,
        grid=(E // blk,),
        in_specs=[
            pl.BlockSpec((blk, D_EDGE), lambda i: (i, 0)),
            pl.BlockSpec((blk, HID2 // 2), lambda i: (i, 0)),
            pl.BlockSpec((blk, HID2 // 2), lambda i: (i, 0)),
            pl.BlockSpec((blk, 1), lambda i: (i, 0)),
            pl.BlockSpec((D_EDGE, HID2), lambda i: (0, 0)),
            pl.BlockSpec((1, HID2), lambda i: (0, 0)),
            pl.BlockSpec((HID2, OUT), lambda i: (0, 0)),
            pl.BlockSpec((1, OUT), lambda i: (0, 0)),
        ],
        out_specs=pl.BlockSpec((blk, OUT), lambda i: (i, 0)),
        out_shape=jax.ShapeDtypeStruct((E, OUT), jnp.float32),
    )(edge_attr, g1, g2, mask2d, w1e, b1, w2, b2)


def _k4_body(dst_hbm, msg_hbm, zeros_hbm, pacc_hbm,
             idx_v, rows_v, acc_spmem):
    c = lax.axis_index("c")
    s = lax.axis_index("s")
    zoff = pl.multiple_of(s * ROWS_PER_TILE, 8)
    pltpu.sync_copy(zeros_hbm, acc_spmem.at[pl.ds(zoff, ROWS_PER_TILE)])
    plsc.subcore_barrier()

    base = pl.multiple_of((c * NS + s) * EPW, 8)

    def step(k, carry):
        off = pl.multiple_of(base + k * CHUNK, 8)
        pltpu.sync_copy(dst_hbm.at[pl.ds(off, CHUNK)], idx_v)
        pltpu.sync_copy(msg_hbm.at[pl.ds(off, CHUNK)], rows_v)
        pltpu.sync_copy(rows_v, acc_spmem.at[idx_v], add=True)
        return carry

    lax.fori_loop(0, NCHUNK, step, 0)
    plsc.subcore_barrier()
    pltpu.sync_copy(acc_spmem.at[pl.ds(zoff, ROWS_PER_TILE)],
                    pacc_hbm.at[c, pl.ds(zoff, ROWS_PER_TILE)])


def _scatter(dst, msg, zeros):
    mesh = plsc.VectorSubcoreMesh(core_axis_name="c", subcore_axis_name="s")
    f = functools.partial(
        pl.kernel,
        out_type=jax.ShapeDtypeStruct((NC, N_PAD, OUT), jnp.float32),
        mesh=mesh,
        scratch_types=[
            pltpu.VMEM((CHUNK,), jnp.int32),
            pltpu.VMEM((CHUNK, OUT), jnp.float32),
            pltpu.VMEM_SHARED((N_PAD, OUT), jnp.float32),
        ],
    )(_k4_body)
    return f(dst, msg, zeros)


def _k5_body(pacc, out):
    p = pacc[...]
    out[...] = jnp.maximum(p[0] + p[1], 0.0)


def _finish(pacc):
    blk = 1000
    return pl.pallas_call(
        _k5_body,
        grid=(N // blk,),
        in_specs=[pl.BlockSpec((NC, blk, OUT), lambda i: (0, i, 0))],
        out_specs=pl.BlockSpec((blk, OUT), lambda i: (i, 0)),
        out_shape=jax.ShapeDtypeStruct((N, OUT), jnp.float32),
    )(pacc)


def kernel(coordinates, edge_attr, src, dst, non_fictitious,
           W1_src, b1_src, W2_src, b2_src,
           W1_dst, b1_dst, W2_dst, b2_dst):
    W1c = jnp.concatenate([W1_src, W1_dst], axis=1)
    b1c = jnp.concatenate([b1_src, b1_dst])[None, :]
    W2c = jnp.concatenate([W2_src, W2_dst], axis=0)
    b2c = (b2_src + b2_dst)[None, :]
    W1e = W1c[:D_EDGE]
    W1a = W1c[D_EDGE:D_EDGE + D]
    W1b = W1c[D_EDGE + D:]

    p1, p2 = _project(coordinates, W1a, W1b)
    g1, g2 = _gather(p1, p2, src, dst)
    msg = _edge_mlp(edge_attr, g1, g2, non_fictitious[:, None],
                    W1e, b1c, W2c, b2c)
    zeros = jnp.zeros((ROWS_PER_TILE, OUT), jnp.float32)
    pacc = _scatter(dst, msg, zeros)
    return _finish(pacc)
